# Initial kernel scaffold; baseline (speedup 1.0000x reference)
import jax
import jax.numpy as jnp
from jax.experimental import pallas as pl


def kernel(x, pe_table):
    raise NotImplementedError("write your pallas kernel here")


# speedup vs baseline: 1.9890x; 1.9890x over previous
import functools

import jax
import jax.numpy as jnp
from jax import lax
from jax.experimental import pallas as pl
from jax.experimental.pallas import tpu as pltpu
from jax.experimental.pallas import tpu_sc as plsc

D_MODEL = 1024
NUM_WORKERS = 32
CHUNK = 32


def _gather_body(table_hbm, idx_hbm, out_hbm, idx_v, rows_v, sem):
    n_idx = idx_hbm.shape[0]
    b_per_w = n_idx // NUM_WORKERS
    wid = lax.axis_index("s") * 2 + lax.axis_index("c")
    base = wid * b_per_w
    pltpu.sync_copy(idx_hbm.at[pl.ds(base, b_per_w)], idx_v)

    def step(c, carry):
        off = c * CHUNK
        pltpu.async_copy(
            table_hbm.at[idx_v.at[pl.ds(off, CHUNK)]], rows_v, sem
        ).wait()
        pltpu.sync_copy(rows_v, out_hbm.at[pl.ds(base + off, CHUNK)])
        return carry

    lax.fori_loop(0, b_per_w // CHUNK, step, 0)


def kernel(x, pe_table):
    batch, seq_len = x.shape
    n = batch * seq_len
    idx = x.reshape(n).astype(jnp.int32)
    mesh = plsc.VectorSubcoreMesh(core_axis_name="c", subcore_axis_name="s")
    gather = functools.partial(
        pl.kernel,
        mesh=mesh,
        out_type=jax.ShapeDtypeStruct((n, D_MODEL), jnp.float32),
        scratch_types=[
            pltpu.VMEM((n // NUM_WORKERS,), jnp.int32),
            pltpu.VMEM((CHUNK, D_MODEL), jnp.float32),
            pltpu.SemaphoreType.DMA,
        ],
    )(_gather_body)
    out = gather(pe_table, idx)
    return out.reshape(batch, seq_len, D_MODEL)


# speedup vs baseline: 2.3707x; 1.1919x over previous
import functools

import jax
import jax.numpy as jnp
from jax import lax
from jax.experimental import pallas as pl
from jax.experimental.pallas import tpu as pltpu
from jax.experimental.pallas import tpu_sc as plsc

D_MODEL = 1024
NUM_WORKERS = 32
CHUNK = 32


def _gather_body(table_hbm, idx_hbm, out_hbm, idx_v, buf0, buf1,
                 g0, g1, w0, w1):
    n_idx = idx_hbm.shape[0]
    b_per_w = n_idx // NUM_WORKERS
    nchunks = b_per_w // CHUNK
    npairs = nchunks // 2
    wid = lax.axis_index("s") * 2 + lax.axis_index("c")
    base = wid * b_per_w
    pltpu.sync_copy(idx_hbm.at[pl.ds(base, b_per_w)], idx_v)

    def g_copy(off, buf, sem):
        return pltpu.make_async_copy(
            table_hbm.at[idx_v.at[pl.ds(off, CHUNK)]], buf, sem)

    def w_copy(off, buf, sem):
        return pltpu.make_async_copy(
            buf, out_hbm.at[pl.ds(base + off, CHUNK)], sem)

    g_copy(0, buf0, g0).start()

    def pair(i, carry):
        a = 2 * i * CHUNK

        @pl.when(i > 0)
        def _():
            w_copy(a - CHUNK, buf1, w1).wait()

        g_copy(a + CHUNK, buf1, g1).start()
        g_copy(a, buf0, g0).wait()
        w_copy(a, buf0, w0).start()

        @pl.when(i < npairs - 1)
        def _():
            w_copy(a, buf0, w0).wait()
            g_copy(a + 2 * CHUNK, buf0, g0).start()

        g_copy(a + CHUNK, buf1, g1).wait()
        w_copy(a + CHUNK, buf1, w1).start()
        return carry

    lax.fori_loop(0, npairs, pair, 0)
    last = (nchunks - 2) * CHUNK
    w_copy(last, buf0, w0).wait()
    w_copy(last + CHUNK, buf1, w1).wait()


def kernel(x, pe_table):
    batch, seq_len = x.shape
    n = batch * seq_len
    idx = x.reshape(n).astype(jnp.int32)
    mesh = plsc.VectorSubcoreMesh(core_axis_name="c", subcore_axis_name="s")
    gather = functools.partial(
        pl.kernel,
        mesh=mesh,
        out_type=jax.ShapeDtypeStruct((n, D_MODEL), jnp.float32),
        scratch_types=[
            pltpu.VMEM((n // NUM_WORKERS,), jnp.int32),
            pltpu.VMEM((CHUNK, D_MODEL), jnp.float32),
            pltpu.VMEM((CHUNK, D_MODEL), jnp.float32),
            pltpu.SemaphoreType.DMA,
            pltpu.SemaphoreType.DMA,
            pltpu.SemaphoreType.DMA,
            pltpu.SemaphoreType.DMA,
        ],
    )(_gather_body)
    out = gather(pe_table, idx)
    return out.reshape(batch, seq_len, D_MODEL)
